# baseline (speedup 1.0000x reference)
import jax
import jax.numpy as jnp
from jax.experimental import pallas as pl
from jax.experimental.pallas import tpu as pltpu

BK = 256


def _body(xi_ref, xj_ref, wi_ref, bi_ref, wj_ref, bj_ref, gi_ref, gj_ref,
          out_ref):
    zi = (
        jnp.dot(xi_ref[...], wi_ref[...], preferred_element_type=jnp.float32)
        + bi_ref[...]
    )
    zj = (
        jnp.dot(xj_ref[...], wj_ref[...], preferred_element_type=jnp.float32)
        + bj_ref[...]
    )
    acc = jnp.dot(gi_ref[...], zi, preferred_element_type=jnp.float32)
    acc += jnp.dot(gj_ref[...], zj, preferred_element_type=jnp.float32)
    out_ref[...] = jnp.maximum(acc, 0.0)


@jax.jit
def kernel(xi, xj, Gi2k, Gj2k, Wi, bi, Wj, bj):
    n_k = Gi2k.shape[0]
    n_i, ci = xi.shape
    n_j, cj = xj.shape
    ck = Wi.shape[1]

    const = lambda shape: pl.BlockSpec(shape, lambda i: (0, 0))
    out = pl.pallas_call(
        _body,
        grid=(n_k // BK,),
        in_specs=[
            const((n_i, ci)),
            const((n_j, cj)),
            const((ci, ck)),
            const((1, ck)),
            const((cj, ck)),
            const((1, ck)),
            pl.BlockSpec((BK, n_i), lambda i: (i, 0)),
            pl.BlockSpec((BK, n_j), lambda i: (i, 0)),
        ],
        out_specs=pl.BlockSpec((BK, ck), lambda i: (i, 0)),
        out_shape=jax.ShapeDtypeStruct((n_k, ck), jnp.float32),
        compiler_params=pltpu.CompilerParams(
            dimension_semantics=("parallel",),
        ),
    )(xi, xj, Wi, bi.reshape(1, ck), Wj, bj.reshape(1, ck), Gi2k, Gj2k)
    return out


# speedup vs baseline: 1.1301x; 1.1301x over previous
import jax
import jax.numpy as jnp
from jax import lax
from jax.experimental import pallas as pl
from jax.experimental.pallas import tpu as pltpu

BK = 256
NBUF = 3


def _body(xi_h, xj_h, gi_h, gj_h, wi_h, bi_h, wj_h, bj_h, out_h,
          xi_v, xj_v, wi_v, bi_v, wj_v, bj_v, zi_v, zj_v, gi_b, gj_b, ob,
          sem_i, sem_j, sem_gi, sem_gj, sem_o):
    nkb = gi_h.shape[0] // BK

    def gi_cp(idx, slot):
        return pltpu.make_async_copy(
            gi_h.at[pl.ds(idx * BK, BK)], gi_b.at[slot], sem_gi.at[slot])

    def gj_cp(idx, slot):
        return pltpu.make_async_copy(
            gj_h.at[pl.ds(idx * BK, BK)], gj_b.at[slot], sem_gj.at[slot])

    def out_cp(idx, slot):
        return pltpu.make_async_copy(
            ob.at[slot], out_h.at[pl.ds(idx * BK, BK)], sem_o.at[slot])

    cp_wi = pltpu.make_async_copy(wi_h, wi_v, sem_i)
    cp_bi = pltpu.make_async_copy(bi_h, bi_v, sem_i)
    cp_xi = pltpu.make_async_copy(xi_h, xi_v, sem_i)
    cp_wi.start(); cp_bi.start(); cp_xi.start()
    gi_cp(0, 0).start()
    cp_wj = pltpu.make_async_copy(wj_h, wj_v, sem_j)
    cp_bj = pltpu.make_async_copy(bj_h, bj_v, sem_j)
    cp_xj = pltpu.make_async_copy(xj_h, xj_v, sem_j)
    cp_wj.start(); cp_bj.start(); cp_xj.start()
    gj_cp(0, 0).start()
    gi_cp(1, 1).start()
    gj_cp(1, 1).start()

    cp_wi.wait(); cp_bi.wait(); cp_xi.wait()
    zi_v[...] = (
        jnp.dot(xi_v[...], wi_v[...], preferred_element_type=jnp.float32)
        + bi_v[...]
    )
    cp_wj.wait(); cp_bj.wait(); cp_xj.wait()
    zj_v[...] = (
        jnp.dot(xj_v[...], wj_v[...], preferred_element_type=jnp.float32)
        + bj_v[...]
    )

    def step(i, carry):
        slot = lax.rem(i, NBUF)
        oslot = lax.rem(i, 2)

        @pl.when(i + 2 < nkb)
        def _prefetch():
            nslot = lax.rem(i + 2, NBUF)
            gi_cp(i + 2, nslot).start()
            gj_cp(i + 2, nslot).start()

        gi_cp(i, slot).wait()
        gj_cp(i, slot).wait()

        @pl.when(i >= 2)
        def _drain_out():
            out_cp(i - 2, oslot).wait()

        acc = jnp.dot(gi_b[slot], zi_v[...], preferred_element_type=jnp.float32)
        acc += jnp.dot(gj_b[slot], zj_v[...], preferred_element_type=jnp.float32)
        ob[oslot] = jnp.maximum(acc, 0.0)
        out_cp(i, oslot).start()
        return carry

    lax.fori_loop(0, nkb, step, 0)
    out_cp(nkb - 2, lax.rem(nkb - 2, 2)).wait()
    out_cp(nkb - 1, lax.rem(nkb - 1, 2)).wait()


@jax.jit
def kernel(xi, xj, Gi2k, Gj2k, Wi, bi, Wj, bj):
    n_k = Gi2k.shape[0]
    n_i, ci = xi.shape
    n_j, cj = xj.shape
    ck = Wi.shape[1]

    any_spec = pl.BlockSpec(memory_space=pl.ANY)
    out = pl.pallas_call(
        _body,
        in_specs=[any_spec] * 8,
        out_specs=any_spec,
        out_shape=jax.ShapeDtypeStruct((n_k, ck), jnp.float32),
        scratch_shapes=[
            pltpu.VMEM((n_i, ci), jnp.float32),
            pltpu.VMEM((n_j, cj), jnp.float32),
            pltpu.VMEM((ci, ck), jnp.float32),
            pltpu.VMEM((1, ck), jnp.float32),
            pltpu.VMEM((cj, ck), jnp.float32),
            pltpu.VMEM((1, ck), jnp.float32),
            pltpu.VMEM((n_i, ck), jnp.float32),
            pltpu.VMEM((n_j, ck), jnp.float32),
            pltpu.VMEM((NBUF, BK, n_i), jnp.float32),
            pltpu.VMEM((NBUF, BK, n_j), jnp.float32),
            pltpu.VMEM((2, BK, ck), jnp.float32),
            pltpu.SemaphoreType.DMA,
            pltpu.SemaphoreType.DMA,
            pltpu.SemaphoreType.DMA((NBUF,)),
            pltpu.SemaphoreType.DMA((NBUF,)),
            pltpu.SemaphoreType.DMA((2,)),
        ],
        compiler_params=pltpu.CompilerParams(
            vmem_limit_bytes=110 * 1024 * 1024,
        ),
    )(xi, xj, Gi2k, Gj2k, Wi, bi.reshape(1, ck), Wj, bj.reshape(1, ck))
    return out


# speedup vs baseline: 1.1449x; 1.0131x over previous
import jax
import jax.numpy as jnp
from jax.experimental import pallas as pl
from jax.experimental.pallas import tpu as pltpu

BK = 256


def _body(xi_ref, xj_ref, wi_ref, bi_ref, wj_ref, bj_ref, gi_ref, gj_ref,
          out_ref, zi_s, zj_s):
    @pl.when(pl.program_id(0) == 0)
    def _init():
        zi_s[...] = (
            jnp.dot(xi_ref[...], wi_ref[...], preferred_element_type=jnp.float32)
            + bi_ref[...]
        )
        zj_s[...] = (
            jnp.dot(xj_ref[...], wj_ref[...], preferred_element_type=jnp.float32)
            + bj_ref[...]
        )

    acc = jnp.dot(gi_ref[...].astype(jnp.bfloat16),
                  zi_s[...].astype(jnp.bfloat16),
                  preferred_element_type=jnp.float32)
    acc += jnp.dot(gj_ref[...].astype(jnp.bfloat16),
                   zj_s[...].astype(jnp.bfloat16),
                   preferred_element_type=jnp.float32)
    out_ref[...] = jnp.maximum(acc, 0.0)


@jax.jit
def kernel(xi, xj, Gi2k, Gj2k, Wi, bi, Wj, bj):
    n_k = Gi2k.shape[0]
    n_i, ci = xi.shape
    n_j, cj = xj.shape
    ck = Wi.shape[1]
    grid = (n_k // BK,)

    const = lambda shape: pl.BlockSpec(shape, lambda i: (0, 0))
    out = pl.pallas_call(
        _body,
        grid=grid,
        in_specs=[
            const((n_i, ci)),
            const((n_j, cj)),
            const((ci, ck)),
            const((1, ck)),
            const((cj, ck)),
            const((1, ck)),
            pl.BlockSpec((BK, n_i), lambda i: (i, 0)),
            pl.BlockSpec((BK, n_j), lambda i: (i, 0)),
        ],
        out_specs=pl.BlockSpec((BK, ck), lambda i: (i, 0)),
        out_shape=jax.ShapeDtypeStruct((n_k, ck), jnp.float32),
        scratch_shapes=[
            pltpu.VMEM((n_i, ck), jnp.float32),
            pltpu.VMEM((n_j, ck), jnp.float32),
        ],
        compiler_params=pltpu.CompilerParams(
            dimension_semantics=("arbitrary",),
        ),
    )(xi, xj, Wi, bi.reshape(1, ck), Wj, bj.reshape(1, ck), Gi2k, Gj2k)
    return out


# speedup vs baseline: 1.1528x; 1.0069x over previous
import jax
import jax.numpy as jnp
from jax.experimental import pallas as pl
from jax.experimental.pallas import tpu as pltpu

BK = 256


def _body(xi_ref, xj_ref, wi_ref, bi_ref, wj_ref, bj_ref, gi_ref, gj_ref,
          out_ref, zi_s, zj_s):
    @pl.when(pl.program_id(0) == 0)
    def _init():
        zi_s[...] = (
            jnp.dot(xi_ref[...], wi_ref[...], preferred_element_type=jnp.float32)
            + bi_ref[...]
        )
        zj_s[...] = (
            jnp.dot(xj_ref[...], wj_ref[...], preferred_element_type=jnp.float32)
            + bj_ref[...]
        )

    acc = jnp.dot(gi_ref[...], zi_s[...], preferred_element_type=jnp.float32)
    acc += jnp.dot(gj_ref[...], zj_s[...], preferred_element_type=jnp.float32)
    out_ref[...] = jnp.maximum(acc, 0.0)


@jax.jit
def kernel(xi, xj, Gi2k, Gj2k, Wi, bi, Wj, bj):
    n_k = Gi2k.shape[0]
    n_i, ci = xi.shape
    n_j, cj = xj.shape
    ck = Wi.shape[1]
    grid = (n_k // BK,)

    const = lambda shape: pl.BlockSpec(shape, lambda i: (0, 0),
                                       pipeline_mode=pl.Buffered(buffer_count=1))
    out = pl.pallas_call(
        _body,
        grid=grid,
        in_specs=[
            const((n_i, ci)),
            const((n_j, cj)),
            const((ci, ck)),
            const((1, ck)),
            const((cj, ck)),
            const((1, ck)),
            pl.BlockSpec((BK, n_i), lambda i: (i, 0),
                         pipeline_mode=pl.Buffered(buffer_count=2)),
            pl.BlockSpec((BK, n_j), lambda i: (i, 0),
                         pipeline_mode=pl.Buffered(buffer_count=2)),
        ],
        out_specs=pl.BlockSpec((BK, ck), lambda i: (i, 0)),
        out_shape=jax.ShapeDtypeStruct((n_k, ck), jnp.float32),
        scratch_shapes=[
            pltpu.VMEM((n_i, ck), jnp.float32),
            pltpu.VMEM((n_j, ck), jnp.float32),
        ],
        compiler_params=pltpu.CompilerParams(
            dimension_semantics=("arbitrary",),
        ),
    )(xi, xj, Wi, bi.reshape(1, ck), Wj, bj.reshape(1, ck), Gi2k, Gj2k)
    return out


# speedup vs baseline: 1.1529x; 1.0000x over previous
import jax
import jax.numpy as jnp
from jax.experimental import pallas as pl
from jax.experimental.pallas import tpu as pltpu

BK = 256


def _body(xi_ref, xj_ref, wi_ref, bi_ref, wj_ref, bj_ref, gi_ref, gj_ref,
          out_ref, zi_s, zj_s):
    @pl.when(pl.program_id(0) == 0)
    def _init():
        zi_s[...] = (
            jnp.dot(xi_ref[...], wi_ref[...], preferred_element_type=jnp.float32)
            + bi_ref[...]
        )
        zj_s[...] = (
            jnp.dot(xj_ref[...], wj_ref[...], preferred_element_type=jnp.float32)
            + bj_ref[...]
        )

    acc = jnp.dot(gi_ref[...], zi_s[...], preferred_element_type=jnp.float32)
    acc += jnp.dot(gj_ref[...], zj_s[...], preferred_element_type=jnp.float32)
    out_ref[...] = jnp.maximum(acc, 0.0)


@jax.jit
def kernel(xi, xj, Gi2k, Gj2k, Wi, bi, Wj, bj):
    n_k = Gi2k.shape[0]
    n_i, ci = xi.shape
    n_j, cj = xj.shape
    ck = Wi.shape[1]
    grid = (n_k // BK,)

    const = lambda shape: pl.BlockSpec(shape, lambda i: (0, 0))
    out = pl.pallas_call(
        _body,
        grid=grid,
        in_specs=[
            const((n_i, ci)),
            const((n_j, cj)),
            const((ci, ck)),
            const((1, ck)),
            const((cj, ck)),
            const((1, ck)),
            pl.BlockSpec((BK, n_i), lambda i: (i, 0)),
            pl.BlockSpec((BK, n_j), lambda i: (i, 0)),
        ],
        out_specs=pl.BlockSpec((BK, ck), lambda i: (i, 0)),
        out_shape=jax.ShapeDtypeStruct((n_k, ck), jnp.float32),
        scratch_shapes=[
            pltpu.VMEM((n_i, ck), jnp.float32),
            pltpu.VMEM((n_j, ck), jnp.float32),
        ],
        compiler_params=pltpu.CompilerParams(
            dimension_semantics=("arbitrary",),
        ),
    )(xi, xj, Wi, bi.reshape(1, ck), Wj, bj.reshape(1, ck), Gi2k, Gj2k)
    return out
